# baseline (speedup 1.0000x reference)
import functools

import jax
import jax.numpy as jnp
from jax import lax
import numpy as np
from jax.experimental import pallas as pl
from jax.experimental.pallas import tpu as pltpu
from jax.experimental.pallas import tpu_sc as plsc

N = 10000
E = 320000
H = 128
TW = 1
LX = 1.0
LY = 1.0
TMAX = 1.0
DT = 0.1

NP = 10240
NC = 2
NS = 16
NW = NC * NS

SUB = 128
NSUB = 8
CH = SUB * NSUB
EPAD = -(-E // (CH * NW)) * (CH * NW)
NCHUNK = EPAD // CH // NW

BE = 2048

_MESH = plsc.VectorSubcoreMesh(core_axis_name="c", subcore_axis_name="s")


def _wid():
    return lax.axis_index("s") * NC + lax.axis_index("c")


@functools.partial(
    pl.kernel, mesh=_MESH,
    out_type=jax.ShapeDtypeStruct((EPAD, H), jnp.float32),
    scratch_types=[
        pltpu.VMEM((NSUB, SUB), jnp.int32),
        pltpu.VMEM((NSUB, SUB), jnp.int32),
        pltpu.VMEM((SUB, H), jnp.float32),
        pltpu.VMEM((SUB, H), jnp.float32),
        pltpu.SemaphoreType.DMA,
        pltpu.SemaphoreType.DMA,
    ],
)
def _sc_gather(a_hbm, b_hbm, dst_hbm, src_hbm, out_hbm,
               didx, sidx, abuf, bbuf, sema, semb):
    w = _wid()

    def chunk(m, _):
        gid = w * NCHUNK + m
        pltpu.sync_copy(dst_hbm.at[pl.ds(gid * NSUB, NSUB)], didx)
        pltpu.sync_copy(src_hbm.at[pl.ds(gid * NSUB, NSUB)], sidx)
        for j in range(NSUB):
            cpa = pltpu.async_copy(a_hbm.at[didx.at[j]], abuf, sema)
            cpb = pltpu.async_copy(b_hbm.at[sidx.at[j]], bbuf, semb)
            cpa.wait()
            cpb.wait()

            def add_row(i, _):
                for jj in range(H // 16):
                    sl = pl.ds(jj * 16, 16)
                    abuf[i, sl] = abuf[i, sl] + bbuf[i, sl]
                return 0

            lax.fori_loop(0, SUB, add_row, 0, unroll=2)
            pltpu.sync_copy(abuf, out_hbm.at[pl.ds(gid * CH + j * SUB, SUB)])
        return 0

    lax.fori_loop(0, NCHUNK, chunk, 0)


NCHT = EPAD // CH // NW


@functools.partial(
    pl.kernel, mesh=_MESH,
    compiler_params=pltpu.CompilerParams(use_tc_tiling_on_sc=False),
    out_type=jax.ShapeDtypeStruct((NC, NP, H), jnp.float32),
    scratch_types=[
        pltpu.VMEM((NSUB, SUB), jnp.int32),
        pltpu.VMEM((SUB,), jnp.int32),
        pltpu.VMEM((SUB, H), jnp.float32),
        pltpu.VMEM_SHARED((NP, H), jnp.float32),
    ],
)
def _sc_scatter(m2_hbm, dst_hbm, zeros_hbm, out_hbm, didx, didx1, mbuf, acc):
    cid = lax.axis_index("c")
    sid = lax.axis_index("s")
    rows = NP // NS
    pltpu.sync_copy(zeros_hbm.at[pl.ds(sid * rows, rows)],
                    acc.at[pl.ds(sid * rows, rows)])
    plsc.subcore_barrier()

    def chunk(m, _):
        gid = (cid * NS + sid) * NCHT + m
        pltpu.sync_copy(dst_hbm.at[pl.ds(gid * NSUB, NSUB)], didx)
        for j in range(NSUB):
            for jj in range(SUB // 16):
                sl = pl.ds(jj * 16, 16)
                didx1[sl] = didx[j, sl]
            pltpu.sync_copy(m2_hbm.at[pl.ds(gid * CH + j * SUB, SUB)], mbuf)
            pltpu.sync_copy(mbuf, acc.at[didx1], add=True)
        return 0

    lax.fori_loop(0, NCHT, chunk, 0)
    plsc.subcore_barrier()
    pltpu.sync_copy(acc.at[pl.ds(sid * rows, rows)],
                    out_hbm.at[cid, pl.ds(sid * rows, rows)])


CW = 16
NCHW = EPAD // CH // NW


@functools.partial(
    pl.kernel, mesh=_MESH,
    compiler_params=pltpu.CompilerParams(use_tc_tiling_on_sc=False),
    out_type=jax.ShapeDtypeStruct((NC, NP, CW), jnp.float32),
    scratch_types=[
        pltpu.VMEM((NSUB, SUB), jnp.int32),
        pltpu.VMEM((SUB,), jnp.int32),
        pltpu.VMEM((SUB, CW), jnp.float32),
        pltpu.VMEM_SHARED((NP, CW), jnp.float32),
    ],
)
def _sc_count(dst_hbm, zeros_hbm, out_hbm, didx, didx1, ones, acc):
    w = _wid()
    cid = lax.axis_index("c")
    sid = lax.axis_index("s")
    rows = NP // NS

    def fill(i, _):
        ones[i, :] = jnp.full((CW,), 1.0, jnp.float32)
        return 0

    lax.fori_loop(0, SUB, fill, 0)
    pltpu.sync_copy(zeros_hbm.at[pl.ds(sid * rows, rows)],
                    acc.at[pl.ds(sid * rows, rows)])
    plsc.subcore_barrier()

    def chunk(m, _):
        gid = w * NCHW + m
        pltpu.sync_copy(dst_hbm.at[pl.ds(gid * NSUB, NSUB)], didx)
        for j in range(NSUB):
            for jj in range(SUB // 16):
                sl = pl.ds(jj * 16, 16)
                didx1[sl] = didx[j, sl]
            pltpu.sync_copy(ones, acc.at[didx1], add=True)
        return 0

    lax.fori_loop(0, NCHW, chunk, 0)
    plsc.subcore_barrier()
    pltpu.sync_copy(acc.at[pl.ds(sid * rows, rows)],
                    out_hbm.at[cid, pl.ds(sid * rows, rows)])


def _edge_mm_body(m1_ref, w_ref, b_ref, o_ref):
    m1 = jnp.maximum(m1_ref[...], 0.0)
    acc = jnp.dot(m1, w_ref[...], preferred_element_type=jnp.float32,
                  precision=jax.lax.Precision.HIGHEST)
    o_ref[...] = jnp.maximum(acc + b_ref[...], 0.0)


def _edge_mm(m1, w, b):
    e = m1.shape[0]
    grid = e // BE
    return pl.pallas_call(
        _edge_mm_body,
        grid=(grid,),
        in_specs=[
            pl.BlockSpec((BE, H), lambda i: (i, 0)),
            pl.BlockSpec((H, H), lambda i: (0, 0)),
            pl.BlockSpec((1, H), lambda i: (0, 0)),
        ],
        out_specs=pl.BlockSpec((BE, H), lambda i: (i, 0)),
        out_shape=jax.ShapeDtypeStruct((e, H), jnp.float32),
    )(m1, w, b.reshape(1, H))


def _bn(x, g, b, eps=1e-5):
    mu = jnp.mean(x, axis=0)
    var = jnp.mean((x - mu) ** 2, axis=0)
    return g * (x - mu) / jnp.sqrt(var + eps) + b


def _conv_mats(conv):
    w1, w2, w3 = conv['w1'], conv['w2'], conv['w3']
    L0 = H
    L1 = (L0 - 16) // 3 + 1
    L2 = (L1 - 12) // 3 + 1
    L3 = (L2 - 8) // 2 + 1
    i0 = jnp.arange(L0)
    t1 = jnp.arange(L1)
    k1 = i0[:, None] - 3 * t1[None, :]
    m1 = (k1 >= 0) & (k1 < 16)
    g1 = jnp.where(m1, k1, 0)
    W1 = jnp.where(m1[:, None, :], w1.transpose(1, 0, 2)[0][:, g1].transpose(1, 0, 2), 0.0)
    W1 = W1.reshape(L0, 4 * L1)
    B1 = jnp.repeat(conv['b1'], L1)
    s2 = jnp.arange(L1)
    t2 = jnp.arange(L2)
    k2 = s2[:, None] - 3 * t2[None, :]
    m2 = (k2 >= 0) & (k2 < 12)
    g2 = jnp.where(m2, k2, 0)
    W2 = jnp.where(m2[None, :, None, :], w2[:, :, g2].transpose(1, 2, 0, 3), 0.0)
    W2 = W2.reshape(4 * L1, 8 * L2)
    B2 = jnp.repeat(conv['b2'], L2)
    s3 = jnp.arange(L2)
    k3 = s3[:, None] - 2 * jnp.arange(L3)[None, :]
    m3 = (k3 >= 0) & (k3 < 8)
    g3 = jnp.where(m3, k3, 0)
    W3 = jnp.where(m3[None, :, None, :], w3[:, :, g3].transpose(1, 2, 0, 3), 0.0)
    W3 = W3.reshape(8 * L2, L3)
    B3 = jnp.repeat(conv['b3'], L3)
    return W1, B1, W2, B2, W3, B3


def kernel(x, pos, edge_index, batch, params):
    with jax.default_matmul_precision("float32"):
        return _kernel_impl(x, pos, edge_index, batch, params)


def _kernel_impl(x, pos, edge_index, batch, params):
    u = x
    pos_x = pos[:, 1:2] / LX
    pos_y = pos[:, 2:3] / LY
    variables = pos[:, 0:1] / TMAX
    node_input = jnp.concatenate([u, pos_x, pos_y, variables], axis=-1)
    p = params['emb']
    h = node_input @ p['w1'] + p['b1']
    h = _bn(h, p['g1'], p['be1'])
    h = jax.nn.relu(h)
    h = h @ p['w2'] + p['b2']
    h = _bn(h, p['g2'], p['be2'])

    npad = EPAD - E
    srcp = jnp.concatenate([edge_index[0], jnp.zeros((npad,), jnp.int32)])
    dstg = jnp.concatenate([edge_index[1], jnp.zeros((npad,), jnp.int32)])
    trash = (jnp.arange(npad, dtype=jnp.int32) % (NP - N)) + N
    dsts = jnp.concatenate([edge_index[1], trash])
    src2d = srcp.reshape(EPAD // SUB, SUB)
    dst2dg = dstg.reshape(EPAD // SUB, SUB)
    dst2ds = dsts.reshape(EPAD // SUB, SUB)
    zeros_nh = jnp.zeros((NP, H), jnp.float32)
    zeros_nc = jnp.zeros((NP, CW), jnp.float32)

    cnt2 = _sc_count(dst2ds, zeros_nc)
    cnt = cnt2[0, :N, 0] + cnt2[1, :N, 0]
    inv_cnt = 1.0 / jnp.maximum(cnt, 1.0)

    for lp in params['layers']:
        wm1 = lp['wm1']
        w_hd = wm1[:H]
        w_hs = wm1[H:2 * H]
        w_u = wm1[2 * H:2 * H + TW]
        w_px = wm1[2 * H + TW:2 * H + TW + 1]
        w_py = wm1[2 * H + TW + 1:2 * H + TW + 2]
        w_v = wm1[2 * H + TW + 2:]
        P = u @ w_u + pos_x @ w_px + pos_y @ w_py
        A = h @ w_hd + P + variables @ w_v + lp['bm1']
        B = h @ w_hs - P
        m1 = _sc_gather(A, B, dst2dg, src2d)
        m2 = _edge_mm(m1, lp['wm2'], lp['bm2'])
        aggp = _sc_scatter(m2, dst2ds, zeros_nh)
        agg = (aggp[0, :N] + aggp[1, :N]) * inv_cnt[:, None]
        wu1 = lp['wu1']
        upd = jax.nn.relu(h @ wu1[:H] + agg @ wu1[H:2 * H]
                          + variables @ wu1[2 * H:] + lp['bu1'])
        upd = jax.nn.relu(upd @ lp['wu2'] + lp['bu2'])
        h = h + upd
        h = _bn(h, lp['g'], lp['be'])

    W1, B1, W2, B2, W3, B3 = _conv_mats(params['conv'])
    z = jax.nn.relu(h @ W1 + B1)
    z = jax.nn.relu(z @ W2 + B2)
    z = z @ W3 + B3
    dt = jnp.cumsum(jnp.ones((1, TW), jnp.float32) * DT * 0.1, axis=1)
    return dt * z


# speedup vs baseline: 1.2340x; 1.2340x over previous
import functools

import jax
import jax.numpy as jnp
from jax import lax
import numpy as np
from jax.experimental import pallas as pl
from jax.experimental.pallas import tpu as pltpu
from jax.experimental.pallas import tpu_sc as plsc

N = 10000
E = 320000
H = 128
TW = 1
LX = 1.0
LY = 1.0
TMAX = 1.0
DT = 0.1

NP = 10240
NC = 2
NS = 16
NW = NC * NS

SUB = 128
NSUB = 8
CH = SUB * NSUB
EPAD = -(-E // (CH * NW)) * (CH * NW)
NCHUNK = EPAD // CH // NW

BE = 2048

_MESH = plsc.VectorSubcoreMesh(core_axis_name="c", subcore_axis_name="s")


def _wid():
    return lax.axis_index("s") * NC + lax.axis_index("c")


@functools.partial(
    pl.kernel, mesh=_MESH,
    out_type=jax.ShapeDtypeStruct((EPAD, H), jnp.float32),
    scratch_types=[
        pltpu.VMEM((NSUB, SUB), jnp.int32),
        pltpu.VMEM((NSUB, SUB), jnp.int32),
        pltpu.VMEM((SUB, H), jnp.float32),
        pltpu.VMEM((SUB, H), jnp.float32),
        pltpu.VMEM((SUB, H), jnp.float32),
        pltpu.VMEM((SUB, H), jnp.float32),
        pltpu.SemaphoreType.DMA,
        pltpu.SemaphoreType.DMA,
        pltpu.SemaphoreType.DMA,
        pltpu.SemaphoreType.DMA,
        pltpu.SemaphoreType.DMA,
        pltpu.SemaphoreType.DMA,
    ],
)
def _sc_gather(a_hbm, b_hbm, dst_hbm, src_hbm, out_hbm,
               didx, sidx, ab0, ab1, bb0, bb1,
               sa0, sa1, sb0, sb1, sw0, sw1):
    w = _wid()
    abufs, bbufs = (ab0, ab1), (bb0, bb1)
    sas, sbs, sws = (sa0, sa1), (sb0, sb1), (sw0, sw1)

    def chunk(m, _):
        gid = w * NCHUNK + m
        pltpu.sync_copy(dst_hbm.at[pl.ds(gid * NSUB, NSUB)], didx)
        pltpu.sync_copy(src_hbm.at[pl.ds(gid * NSUB, NSUB)], sidx)

        def fire(j):
            sl = j % 2
            pltpu.async_copy(a_hbm.at[didx.at[j]], abufs[sl], sas[sl])
            pltpu.async_copy(b_hbm.at[sidx.at[j]], bbufs[sl], sbs[sl])

        def drain_gather(j):
            sl = j % 2
            pltpu.make_async_copy(a_hbm.at[didx.at[j]], abufs[sl],
                                  sas[sl]).wait()
            pltpu.make_async_copy(b_hbm.at[sidx.at[j]], bbufs[sl],
                                  sbs[sl]).wait()

        fire(0)
        for j in range(NSUB):
            sl = j % 2
            if j + 1 < NSUB:
                if j + 1 >= 2:
                    pltpu.make_async_copy(
                        abufs[1 - sl],
                        out_hbm.at[pl.ds(gid * CH + (j - 1) * SUB, SUB)],
                        sws[1 - sl]).wait()
                fire(j + 1)
            drain_gather(j)
            ab, bb = abufs[sl], bbufs[sl]

            def add_row(i, _):
                for jj in range(H // 16):
                    c = pl.ds(jj * 16, 16)
                    ab[i, c] = ab[i, c] + bb[i, c]
                return 0

            lax.fori_loop(0, SUB, add_row, 0, unroll=2)
            pltpu.async_copy(ab, out_hbm.at[pl.ds(gid * CH + j * SUB, SUB)],
                             sws[sl])
        for j in (NSUB - 2, NSUB - 1):
            sl = j % 2
            pltpu.make_async_copy(
                abufs[sl], out_hbm.at[pl.ds(gid * CH + j * SUB, SUB)],
                sws[sl]).wait()
        return 0

    lax.fori_loop(0, NCHUNK, chunk, 0)


NCHT = EPAD // CH // NW


@functools.partial(
    pl.kernel, mesh=_MESH,
    compiler_params=pltpu.CompilerParams(use_tc_tiling_on_sc=False),
    out_type=jax.ShapeDtypeStruct((NC, NP, H), jnp.float32),
    scratch_types=[
        pltpu.VMEM((NSUB, SUB), jnp.int32),
        pltpu.VMEM((SUB,), jnp.int32),
        pltpu.VMEM((SUB,), jnp.int32),
        pltpu.VMEM((SUB, H), jnp.float32),
        pltpu.VMEM((SUB, H), jnp.float32),
        pltpu.VMEM_SHARED((NP, H), jnp.float32),
        pltpu.SemaphoreType.DMA,
        pltpu.SemaphoreType.DMA,
        pltpu.SemaphoreType.DMA,
        pltpu.SemaphoreType.DMA,
    ],
)
def _sc_scatter(m2_hbm, dst_hbm, zeros_hbm, out_hbm,
                didx, dx0, dx1, mb0, mb1, acc, sl0, sl1, ss0, ss1):
    cid = lax.axis_index("c")
    sid = lax.axis_index("s")
    rows = NP // NS
    pltpu.sync_copy(zeros_hbm.at[pl.ds(sid * rows, rows)],
                    acc.at[pl.ds(sid * rows, rows)])
    plsc.subcore_barrier()
    mbufs, dxs = (mb0, mb1), (dx0, dx1)
    sls, sss = (sl0, sl1), (ss0, ss1)

    def chunk(m, _):
        gid = (cid * NS + sid) * NCHT + m
        pltpu.sync_copy(dst_hbm.at[pl.ds(gid * NSUB, NSUB)], didx)

        def fire_load(j):
            sl = j % 2
            pltpu.async_copy(m2_hbm.at[pl.ds(gid * CH + j * SUB, SUB)],
                             mbufs[sl], sls[sl])

        fire_load(0)
        for j in range(NSUB):
            sl = j % 2
            if j + 1 < NSUB:
                if j + 1 >= 2:
                    pltpu.make_async_copy(mbufs[1 - sl],
                                          acc.at[dxs[1 - sl]],
                                          sss[1 - sl]).wait()
                fire_load(j + 1)
            for jj in range(SUB // 16):
                c = pl.ds(jj * 16, 16)
                dxs[sl][c] = didx[j, c]
            pltpu.make_async_copy(
                m2_hbm.at[pl.ds(gid * CH + j * SUB, SUB)],
                mbufs[sl], sls[sl]).wait()
            pltpu.async_copy(mbufs[sl], acc.at[dxs[sl]], sss[sl],
                             add=True)
        for j in (NSUB - 2, NSUB - 1):
            sl = j % 2
            pltpu.make_async_copy(mbufs[sl], acc.at[dxs[sl]],
                                  sss[sl]).wait()
        return 0

    lax.fori_loop(0, NCHT, chunk, 0)
    plsc.subcore_barrier()
    pltpu.sync_copy(acc.at[pl.ds(sid * rows, rows)],
                    out_hbm.at[cid, pl.ds(sid * rows, rows)])


CW = 16
NCHW = EPAD // CH // NW


@functools.partial(
    pl.kernel, mesh=_MESH,
    compiler_params=pltpu.CompilerParams(use_tc_tiling_on_sc=False),
    out_type=jax.ShapeDtypeStruct((NC, NP, CW), jnp.float32),
    scratch_types=[
        pltpu.VMEM((NSUB, SUB), jnp.int32),
        pltpu.VMEM((SUB,), jnp.int32),
        pltpu.VMEM((SUB, CW), jnp.float32),
        pltpu.VMEM_SHARED((NP, CW), jnp.float32),
    ],
)
def _sc_count(dst_hbm, zeros_hbm, out_hbm, didx, didx1, ones, acc):
    w = _wid()
    cid = lax.axis_index("c")
    sid = lax.axis_index("s")
    rows = NP // NS

    def fill(i, _):
        ones[i, :] = jnp.full((CW,), 1.0, jnp.float32)
        return 0

    lax.fori_loop(0, SUB, fill, 0)
    pltpu.sync_copy(zeros_hbm.at[pl.ds(sid * rows, rows)],
                    acc.at[pl.ds(sid * rows, rows)])
    plsc.subcore_barrier()

    def chunk(m, _):
        gid = w * NCHW + m
        pltpu.sync_copy(dst_hbm.at[pl.ds(gid * NSUB, NSUB)], didx)
        for j in range(NSUB):
            for jj in range(SUB // 16):
                sl = pl.ds(jj * 16, 16)
                didx1[sl] = didx[j, sl]
            pltpu.sync_copy(ones, acc.at[didx1], add=True)
        return 0

    lax.fori_loop(0, NCHW, chunk, 0)
    plsc.subcore_barrier()
    pltpu.sync_copy(acc.at[pl.ds(sid * rows, rows)],
                    out_hbm.at[cid, pl.ds(sid * rows, rows)])


def _edge_mm_body(m1_ref, w_ref, b_ref, o_ref):
    m1 = jnp.maximum(m1_ref[...], 0.0)
    acc = jnp.dot(m1, w_ref[...], preferred_element_type=jnp.float32,
                  precision=jax.lax.Precision.HIGHEST)
    o_ref[...] = jnp.maximum(acc + b_ref[...], 0.0)


def _edge_mm(m1, w, b):
    e = m1.shape[0]
    grid = e // BE
    return pl.pallas_call(
        _edge_mm_body,
        grid=(grid,),
        in_specs=[
            pl.BlockSpec((BE, H), lambda i: (i, 0)),
            pl.BlockSpec((H, H), lambda i: (0, 0)),
            pl.BlockSpec((1, H), lambda i: (0, 0)),
        ],
        out_specs=pl.BlockSpec((BE, H), lambda i: (i, 0)),
        out_shape=jax.ShapeDtypeStruct((e, H), jnp.float32),
    )(m1, w, b.reshape(1, H))


def _bn(x, g, b, eps=1e-5):
    mu = jnp.mean(x, axis=0)
    var = jnp.mean((x - mu) ** 2, axis=0)
    return g * (x - mu) / jnp.sqrt(var + eps) + b


def _conv_mats(conv):
    w1, w2, w3 = conv['w1'], conv['w2'], conv['w3']
    L0 = H
    L1 = (L0 - 16) // 3 + 1
    L2 = (L1 - 12) // 3 + 1
    L3 = (L2 - 8) // 2 + 1
    i0 = jnp.arange(L0)
    t1 = jnp.arange(L1)
    k1 = i0[:, None] - 3 * t1[None, :]
    m1 = (k1 >= 0) & (k1 < 16)
    g1 = jnp.where(m1, k1, 0)
    W1 = jnp.where(m1[:, None, :], w1.transpose(1, 0, 2)[0][:, g1].transpose(1, 0, 2), 0.0)
    W1 = W1.reshape(L0, 4 * L1)
    B1 = jnp.repeat(conv['b1'], L1)
    s2 = jnp.arange(L1)
    t2 = jnp.arange(L2)
    k2 = s2[:, None] - 3 * t2[None, :]
    m2 = (k2 >= 0) & (k2 < 12)
    g2 = jnp.where(m2, k2, 0)
    W2 = jnp.where(m2[None, :, None, :], w2[:, :, g2].transpose(1, 2, 0, 3), 0.0)
    W2 = W2.reshape(4 * L1, 8 * L2)
    B2 = jnp.repeat(conv['b2'], L2)
    s3 = jnp.arange(L2)
    k3 = s3[:, None] - 2 * jnp.arange(L3)[None, :]
    m3 = (k3 >= 0) & (k3 < 8)
    g3 = jnp.where(m3, k3, 0)
    W3 = jnp.where(m3[None, :, None, :], w3[:, :, g3].transpose(1, 2, 0, 3), 0.0)
    W3 = W3.reshape(8 * L2, L3)
    B3 = jnp.repeat(conv['b3'], L3)
    return W1, B1, W2, B2, W3, B3


def kernel(x, pos, edge_index, batch, params):
    with jax.default_matmul_precision("float32"):
        return _kernel_impl(x, pos, edge_index, batch, params)


def _kernel_impl(x, pos, edge_index, batch, params):
    u = x
    pos_x = pos[:, 1:2] / LX
    pos_y = pos[:, 2:3] / LY
    variables = pos[:, 0:1] / TMAX
    node_input = jnp.concatenate([u, pos_x, pos_y, variables], axis=-1)
    p = params['emb']
    h = node_input @ p['w1'] + p['b1']
    h = _bn(h, p['g1'], p['be1'])
    h = jax.nn.relu(h)
    h = h @ p['w2'] + p['b2']
    h = _bn(h, p['g2'], p['be2'])

    npad = EPAD - E
    srcp = jnp.concatenate([edge_index[0], jnp.zeros((npad,), jnp.int32)])
    dstg = jnp.concatenate([edge_index[1], jnp.zeros((npad,), jnp.int32)])
    trash = (jnp.arange(npad, dtype=jnp.int32) % (NP - N)) + N
    dsts = jnp.concatenate([edge_index[1], trash])
    src2d = srcp.reshape(EPAD // SUB, SUB)
    dst2dg = dstg.reshape(EPAD // SUB, SUB)
    dst2ds = dsts.reshape(EPAD // SUB, SUB)
    zeros_nh = jnp.zeros((NP, H), jnp.float32)
    zeros_nc = jnp.zeros((NP, CW), jnp.float32)

    cnt2 = _sc_count(dst2ds, zeros_nc)
    cnt = cnt2[0, :N, 0] + cnt2[1, :N, 0]
    inv_cnt = 1.0 / jnp.maximum(cnt, 1.0)

    for lp in params['layers']:
        wm1 = lp['wm1']
        w_hd = wm1[:H]
        w_hs = wm1[H:2 * H]
        w_u = wm1[2 * H:2 * H + TW]
        w_px = wm1[2 * H + TW:2 * H + TW + 1]
        w_py = wm1[2 * H + TW + 1:2 * H + TW + 2]
        w_v = wm1[2 * H + TW + 2:]
        P = u @ w_u + pos_x @ w_px + pos_y @ w_py
        A = h @ w_hd + P + variables @ w_v + lp['bm1']
        B = h @ w_hs - P
        m1 = _sc_gather(A, B, dst2dg, src2d)
        m2 = _edge_mm(m1, lp['wm2'], lp['bm2'])
        aggp = _sc_scatter(m2, dst2ds, zeros_nh)
        agg = (aggp[0, :N] + aggp[1, :N]) * inv_cnt[:, None]
        wu1 = lp['wu1']
        upd = jax.nn.relu(h @ wu1[:H] + agg @ wu1[H:2 * H]
                          + variables @ wu1[2 * H:] + lp['bu1'])
        upd = jax.nn.relu(upd @ lp['wu2'] + lp['bu2'])
        h = h + upd
        h = _bn(h, lp['g'], lp['be'])

    W1, B1, W2, B2, W3, B3 = _conv_mats(params['conv'])
    z = jax.nn.relu(h @ W1 + B1)
    z = jax.nn.relu(z @ W2 + B2)
    z = z @ W3 + B3
    dt = jnp.cumsum(jnp.ones((1, TW), jnp.float32) * DT * 0.1, axis=1)
    return dt * z
